# baseline (speedup 1.0000x reference)
import functools

import jax
import jax.numpy as jnp
from jax import lax
from jax.experimental import pallas as pl
from jax.experimental.pallas import tpu as pltpu
from jax.experimental.pallas import tpu_sc as plsc

N_NEIGH = 16
N_CLASSES = 100
TILE_N = 2048
GRP = 128
NSEL = 24

SC_NC = 2
SC_NS = 16
LANES = 16
N_WORKERS = SC_NC * SC_NS


def _dist_body(xtest_ref, xtrain_ref, d2_ref, gm_ref):
    xq = xtest_ref[...]
    xt = xtrain_ref[...]
    q2 = jnp.sum(xq * xq, axis=1, keepdims=True)
    k2 = jnp.sum(xt * xt, axis=1, keepdims=True).T
    mm = lax.dot_general(
        xq, xt,
        dimension_numbers=(((1,), (1,)), ((), ())),
        preferred_element_type=jnp.float32,
    )
    d2 = (q2 + k2) - 2.0 * mm
    gms = []
    for j in range(TILE_N // GRP):
        blk = d2[:, j * GRP:(j + 1) * GRP]
        d2_ref[j] = blk
        gms.append(jnp.min(blk, axis=1, keepdims=True))
    gm_ref[0] = jnp.concatenate(gms, axis=1)


def _distance_matrix(x_train, x_test):
    n, d = x_train.shape
    q = x_test.shape[0]
    n_pad = ((n + TILE_N - 1) // TILE_N) * TILE_N
    if n_pad != n:
        x_train = jnp.pad(x_train, ((0, n_pad - n), (0, 0)),
                          constant_values=1e18)
    grid = n_pad // TILE_N
    g_tile = TILE_N // GRP
    n_grp = n_pad // GRP
    return pl.pallas_call(
        _dist_body,
        grid=(grid,),
        in_specs=[
            pl.BlockSpec((q, d), lambda t: (0, 0)),
            pl.BlockSpec((TILE_N, d), lambda t: (t, 0)),
        ],
        out_specs=[
            pl.BlockSpec((g_tile, q, GRP), lambda t: (t, 0, 0)),
            pl.BlockSpec((1, q, g_tile), lambda t: (t, 0, 0)),
        ],
        out_shape=[
            jax.ShapeDtypeStruct((n_grp, q, GRP), jnp.float32),
            jax.ShapeDtypeStruct((grid, q, g_tile), jnp.float32),
        ],
    )(x_test, x_train)


def _select_body(gm_ref, ids_ref, mins_ref):
    g = gm_ref[...]
    col = lax.broadcasted_iota(jnp.int32, g.shape, 1)
    inf = jnp.float32(jnp.inf)
    ids, mins = [], []
    for _ in range(NSEL):
        m = jnp.min(g, axis=1, keepdims=True)
        pick = jnp.where(g == m, col, jnp.int32(2**30))
        idx = jnp.min(pick, axis=1, keepdims=True)
        ids.append(idx)
        mins.append(m)
        g = jnp.where(col == idx, inf, g)
    ids_ref[...] = jnp.concatenate(ids, axis=1)
    mins_ref[...] = jnp.concatenate(mins, axis=1)


def _select_groups(gmins):
    q, n_grp = gmins.shape
    return pl.pallas_call(
        _select_body,
        out_shape=[
            jax.ShapeDtypeStruct((q, NSEL), jnp.int32),
            jax.ShapeDtypeStruct((q, NSEL), jnp.float32),
        ],
    )(gmins)


def _merge16(bd, bi, vd, vi):
    sd, si = plsc.sort_key_val(vd, vi, descending=True)
    sel = sd < bd
    nd = jnp.where(sel, sd, bd)
    ni = jnp.where(sel, si, bi)
    return plsc.sort_key_val(nd, ni)


def _bcast_lane(v, lane):
    idx = jnp.full((LANES,), lane, jnp.int32)
    dn = lax.GatherDimensionNumbers(offset_dims=(), collapsed_slice_dims=(0,),
                                    start_index_map=(0,))
    return lax.gather(v, idx[:, None], dn, (1,),
                      mode=lax.GatherScatterMode.PROMISE_IN_BOUNDS)


def _sc_topk_body(n_q, qpw, d2f_hbm, gids_hbm, selm_hbm, y_hbm, out_hbm,
                  y_ref, gid_ref, selm_ref, grp_ref, res_ref, sem):
    c_id = lax.axis_index("c")
    s_id = lax.axis_index("s")
    wid = s_id * SC_NC + c_id
    q0 = wid * qpw
    iota = lax.iota(jnp.int32, LANES)
    lane0 = iota == 0
    zidx = jnp.zeros((LANES,), jnp.int32)
    inf = jnp.float32(jnp.inf)
    pltpu.sync_copy(y_hbm, y_ref)
    pltpu.sync_copy(gids_hbm.at[pl.ds(q0, qpw)], gid_ref)
    pltpu.sync_copy(selm_hbm.at[pl.ds(q0, qpw)], selm_ref)

    def fire(j, b):
        jv = jnp.full((LANES,), j, jnp.int32)
        g0 = plsc.load_gather(gid_ref, [jv, iota])
        g1 = plsc.load_gather(gid_ref, [jv, iota + 8])
        idx0 = g0 * n_q + (q0 + j)
        idx1 = g1 * n_q + (q0 + j)
        pltpu.async_copy(d2f_hbm.at[idx0], grp_ref.at[b, pl.ds(0, LANES)], sem)
        pltpu.async_copy(d2f_hbm.at[idx1], grp_ref.at[b, pl.ds(8, LANES)], sem)

    fire(0, 0)

    def per_query(i, _):
        b = lax.rem(i, 2)
        pltpu.make_async_copy(d2f_hbm.at[zidx],
                              grp_ref.at[b, pl.ds(0, LANES)], sem).wait()
        pltpu.make_async_copy(d2f_hbm.at[zidx],
                              grp_ref.at[b, pl.ds(8, LANES)], sem).wait()

        @pl.when(i + 1 < qpw)
        def _():
            fire(i + 1, 1 - b)

        iv = jnp.full((LANES,), i, jnp.int32)
        bv = jnp.full((LANES,), b, jnp.int32)

        def per_group(k, carry):
            bd0, bi0, thr0 = carry
            kv = jnp.full((LANES,), k, jnp.int32)
            gmn = jnp.min(plsc.load_gather(selm_ref, [iv, kv]))

            def scan_group(bd, bi, thr):
                gidv = plsc.load_gather(gid_ref, [iv, kv])

                def per_blk(j, carry):
                    bd, bi, thr = carry
                    cols = j * LANES + iota
                    v = plsc.load_gather(grp_ref, [bv, kv, cols])
                    mn = jnp.min(v)

                    def slow(bd, bi, _):
                        vi = gidv * GRP + cols
                        bd, bi = _merge16(bd, bi, v, vi)
                        return bd, bi, jnp.max(bd)

                    return lax.cond(mn < thr, slow,
                                    lambda a, c, t: (a, c, t), bd, bi, thr)

                return lax.fori_loop(0, GRP // LANES, per_blk, (bd, bi, thr))

            return lax.cond(gmn < thr0, scan_group,
                            lambda a, c, t: (a, c, t), bd0, bi0, thr0)

        best_d, best_i, _ = lax.fori_loop(
            0, NSEL, per_group,
            (jnp.full((LANES,), inf), jnp.zeros((LANES,), jnp.int32), inf))

        labels = plsc.load_gather(y_ref, [best_i])
        acc = jnp.zeros((LANES,), jnp.int32)
        for j in range(N_NEIGH):
            bc = _bcast_lane(labels, j)
            acc = acc + jnp.where(labels == bc, 1, 0)
        score = acc * 128 - labels
        mx = jnp.max(score)
        win = jnp.max(jnp.where(score == mx, labels, -1))
        plsc.store_scatter(res_ref, [iv], jnp.full((LANES,), win, jnp.int32),
                           mask=lane0)
        return 0

    lax.fori_loop(0, qpw, per_query, 0)
    pltpu.sync_copy(res_ref, out_hbm.at[pl.ds(q0, qpw)])


def _sc_topk(d2f, gids, selm, y32, n_q):
    q = gids.shape[0]
    qpw = q // N_WORKERS
    mesh = plsc.VectorSubcoreMesh(core_axis_name="c", subcore_axis_name="s",
                                  num_cores=SC_NC, num_subcores=SC_NS)
    fn = pl.kernel(
        functools.partial(_sc_topk_body, n_q, qpw),
        out_type=jax.ShapeDtypeStruct((q,), jnp.int32),
        mesh=mesh,
        scratch_types=[
            pltpu.VMEM((y32.shape[0],), jnp.int32),
            pltpu.VMEM((qpw, NSEL), jnp.int32),
            pltpu.VMEM((qpw, NSEL), jnp.float32),
            pltpu.VMEM((2, NSEL, GRP), jnp.float32),
            pltpu.VMEM((qpw,), jnp.int32),
            pltpu.SemaphoreType.DMA,
        ],
        compiler_params=pltpu.CompilerParams(needs_layout_passes=False),
    )
    return fn(d2f, gids, selm, y32)


def kernel(x_train, y_train, x_test):
    n = x_train.shape[0]
    q = x_test.shape[0]
    d2g, gmins3 = _distance_matrix(x_train, x_test)
    n_grp = d2g.shape[0]
    gmins = jnp.transpose(gmins3, (1, 0, 2)).reshape(q, n_grp)
    gids, selm = _select_groups(gmins)
    d2f = d2g.reshape(n_grp * q, GRP)
    y32 = jnp.pad(y_train.astype(jnp.int32), (0, n_grp * GRP - n))
    y_pred = _sc_topk(d2f, gids, selm, y32, q)
    return y_pred.astype(jnp.int64)


# speedup vs baseline: 1.0394x; 1.0394x over previous
import functools

import jax
import jax.numpy as jnp
from jax import lax
from jax.experimental import pallas as pl
from jax.experimental.pallas import tpu as pltpu
from jax.experimental.pallas import tpu_sc as plsc

N_NEIGH = 16
N_CLASSES = 100
TILE_N = 2048
GRP = 128
NSEL = 24

SC_NC = 2
SC_NS = 16
LANES = 16
N_WORKERS = SC_NC * SC_NS


def _dist_body(n_real, xtest_ref, xtrain_ref, d2_ref, gm_ref):
    t = pl.program_id(0)
    xq = xtest_ref[...]
    xt = xtrain_ref[...]
    q2 = jnp.sum(xq * xq, axis=1, keepdims=True)
    k2 = jnp.sum(xt * xt, axis=1, keepdims=True).T
    mm = lax.dot_general(
        xq, xt,
        dimension_numbers=(((1,), (1,)), ((), ())),
        preferred_element_type=jnp.float32,
    )
    d2 = (q2 + k2) - 2.0 * mm
    col = t * TILE_N + lax.broadcasted_iota(jnp.int32, d2.shape, 1)
    d2 = jnp.where(col < n_real, d2, jnp.float32(jnp.inf))
    gms = []
    for j in range(TILE_N // GRP):
        blk = d2[:, j * GRP:(j + 1) * GRP]
        d2_ref[j] = blk
        gms.append(jnp.min(blk, axis=1, keepdims=True))
    gm_ref[0] = jnp.concatenate(gms, axis=1)


def _distance_matrix(x_train, x_test):
    n, d = x_train.shape
    q = x_test.shape[0]
    n_pad = ((n + TILE_N - 1) // TILE_N) * TILE_N
    if n_pad != n:
        x_train = jnp.pad(x_train, ((0, n_pad - n), (0, 0)))
    grid = n_pad // TILE_N
    g_tile = TILE_N // GRP
    n_grp = n_pad // GRP
    return pl.pallas_call(
        functools.partial(_dist_body, n),
        grid=(grid,),
        in_specs=[
            pl.BlockSpec((q, d), lambda t: (0, 0)),
            pl.BlockSpec((TILE_N, d), lambda t: (t, 0)),
        ],
        out_specs=[
            pl.BlockSpec((g_tile, q, GRP), lambda t: (t, 0, 0)),
            pl.BlockSpec((1, q, g_tile), lambda t: (t, 0, 0)),
        ],
        out_shape=[
            jax.ShapeDtypeStruct((n_grp, q, GRP), jnp.float32),
            jax.ShapeDtypeStruct((grid, q, g_tile), jnp.float32),
        ],
    )(x_test, x_train)


def _select_body(gm_ref, ids_ref, mins_ref):
    g = gm_ref[...]
    col = lax.broadcasted_iota(jnp.int32, g.shape, 1)
    inf = jnp.float32(jnp.inf)
    ids, mins = [], []
    for _ in range(NSEL):
        m = jnp.min(g, axis=1, keepdims=True)
        pick = jnp.where(g == m, col, jnp.int32(2**30))
        idx = jnp.min(pick, axis=1, keepdims=True)
        ids.append(idx)
        mins.append(m)
        g = jnp.where(col == idx, inf, g)
    ids_ref[...] = jnp.concatenate(ids, axis=1)
    mins_ref[...] = jnp.concatenate(mins, axis=1)


def _select_groups(gmins):
    q, n_grp = gmins.shape
    return pl.pallas_call(
        _select_body,
        out_shape=[
            jax.ShapeDtypeStruct((q, NSEL), jnp.int32),
            jax.ShapeDtypeStruct((q, NSEL), jnp.float32),
        ],
    )(gmins)


def _merge16(bd, bi, vd, vi):
    sd, si = plsc.sort_key_val(vd, vi, descending=True)
    sel = sd < bd
    nd = jnp.where(sel, sd, bd)
    ni = jnp.where(sel, si, bi)
    return plsc.sort_key_val(nd, ni)


def _bcast_lane(v, lane):
    idx = jnp.full((LANES,), lane, jnp.int32)
    dn = lax.GatherDimensionNumbers(offset_dims=(), collapsed_slice_dims=(0,),
                                    start_index_map=(0,))
    return lax.gather(v, idx[:, None], dn, (1,),
                      mode=lax.GatherScatterMode.PROMISE_IN_BOUNDS)


def _sc_topk_body(n_q, qpw, d2f_hbm, gids_hbm, selm_hbm, y_hbm, out_hbm,
                  y_ref, gid_ref, selm_ref, grp_ref, res_ref, sem):
    c_id = lax.axis_index("c")
    s_id = lax.axis_index("s")
    wid = s_id * SC_NC + c_id
    q0 = wid * qpw
    iota = lax.iota(jnp.int32, LANES)
    lane0 = iota == 0
    zidx = jnp.zeros((LANES,), jnp.int32)
    inf = jnp.float32(jnp.inf)
    pltpu.sync_copy(y_hbm, y_ref)
    pltpu.sync_copy(gids_hbm.at[pl.ds(q0, qpw)], gid_ref)
    pltpu.sync_copy(selm_hbm.at[pl.ds(q0, qpw)], selm_ref)

    def fire(j, b):
        jv = jnp.full((LANES,), j, jnp.int32)
        g0 = plsc.load_gather(gid_ref, [jv, iota])
        g1 = plsc.load_gather(gid_ref, [jv, iota + 8])
        idx0 = g0 * n_q + (q0 + j)
        idx1 = g1 * n_q + (q0 + j)
        pltpu.async_copy(d2f_hbm.at[idx0], grp_ref.at[b, pl.ds(0, LANES)], sem)
        pltpu.async_copy(d2f_hbm.at[idx1], grp_ref.at[b, pl.ds(8, LANES)], sem)

    fire(0, 0)

    def per_query(i, _):
        b = lax.rem(i, 2)
        pltpu.make_async_copy(d2f_hbm.at[zidx],
                              grp_ref.at[b, pl.ds(0, LANES)], sem).wait()
        pltpu.make_async_copy(d2f_hbm.at[zidx],
                              grp_ref.at[b, pl.ds(8, LANES)], sem).wait()

        @pl.when(i + 1 < qpw)
        def _():
            fire(i + 1, 1 - b)

        iv = jnp.full((LANES,), i, jnp.int32)
        bv = jnp.full((LANES,), b, jnp.int32)

        def per_group(k, carry):
            bd0, bi0, thr0 = carry
            kv = jnp.full((LANES,), k, jnp.int32)
            gmn = jnp.min(plsc.load_gather(selm_ref, [iv, kv]))

            def scan_group(bd, bi, thr):
                gidv = plsc.load_gather(gid_ref, [iv, kv])

                def per_blk(j, carry):
                    bd, bi, thr = carry
                    cols = j * LANES + iota
                    v = plsc.load_gather(grp_ref, [bv, kv, cols])
                    mn = jnp.min(v)

                    def slow(bd, bi, _):
                        vi = gidv * GRP + cols
                        bd, bi = _merge16(bd, bi, v, vi)
                        return bd, bi, jnp.max(bd)

                    return lax.cond(mn < thr, slow,
                                    lambda a, c, t: (a, c, t), bd, bi, thr)

                return lax.fori_loop(0, GRP // LANES, per_blk, (bd, bi, thr))

            return lax.cond(gmn < thr0, scan_group,
                            lambda a, c, t: (a, c, t), bd0, bi0, thr0)

        best_d, best_i, _ = lax.fori_loop(
            0, NSEL, per_group,
            (jnp.full((LANES,), inf), jnp.zeros((LANES,), jnp.int32), inf))

        labels = plsc.load_gather(y_ref, [best_i])
        acc = jnp.zeros((LANES,), jnp.int32)
        for j in range(N_NEIGH):
            bc = _bcast_lane(labels, j)
            acc = acc + jnp.where(labels == bc, 1, 0)
        score = acc * 128 - labels
        mx = jnp.max(score)
        win = jnp.max(jnp.where(score == mx, labels, -1))
        plsc.store_scatter(res_ref, [iv], jnp.full((LANES,), win, jnp.int32),
                           mask=lane0)
        return 0

    lax.fori_loop(0, qpw, per_query, 0)
    pltpu.sync_copy(res_ref, out_hbm.at[pl.ds(q0, qpw)])


def _sc_topk(d2f, gids, selm, y32, n_q):
    q = gids.shape[0]
    qpw = q // N_WORKERS
    mesh = plsc.VectorSubcoreMesh(core_axis_name="c", subcore_axis_name="s",
                                  num_cores=SC_NC, num_subcores=SC_NS)
    fn = pl.kernel(
        functools.partial(_sc_topk_body, n_q, qpw),
        out_type=jax.ShapeDtypeStruct((q,), jnp.int32),
        mesh=mesh,
        scratch_types=[
            pltpu.VMEM((y32.shape[0],), jnp.int32),
            pltpu.VMEM((qpw, NSEL), jnp.int32),
            pltpu.VMEM((qpw, NSEL), jnp.float32),
            pltpu.VMEM((2, NSEL, GRP), jnp.float32),
            pltpu.VMEM((qpw,), jnp.int32),
            pltpu.SemaphoreType.DMA,
        ],
        compiler_params=pltpu.CompilerParams(needs_layout_passes=False),
    )
    return fn(d2f, gids, selm, y32)


def kernel(x_train, y_train, x_test):
    n = x_train.shape[0]
    q = x_test.shape[0]
    d2g, gmins3 = _distance_matrix(x_train, x_test)
    n_grp = d2g.shape[0]
    gmins = jnp.transpose(gmins3, (1, 0, 2)).reshape(q, n_grp)
    gids, selm = _select_groups(gmins)
    d2f = d2g.reshape(n_grp * q, GRP)
    y32 = jnp.pad(y_train.astype(jnp.int32), (0, n_grp * GRP - n))
    y_pred = _sc_topk(d2f, gids, selm, y32, q)
    return y_pred.astype(jnp.int64)
